# Initial kernel scaffold; baseline (speedup 1.0000x reference)
import jax
import jax.numpy as jnp
from jax.experimental import pallas as pl


def kernel(x, edge_index, edge_weight, W1, b1, W2, b2):
    raise NotImplementedError("write your pallas kernel here")


# speedup vs baseline: 6.0202x; 6.0202x over previous
import functools

import jax
import jax.numpy as jnp
from jax import lax
from jax.experimental import pallas as pl
from jax.experimental.pallas import tpu as pltpu
from jax.experimental.pallas import tpu_sc as plsc

N = 10000
E = 320000
D0, D1, D2 = 128, 256, 128

NC, NS, L = 2, 16, 16
CH = 128
E_PAD = 327680
DEG_CHUNKS = E_PAD // (NC * NS) // CH
MP_CHUNKS = E_PAD // NS // CH
RPT = N // NS

_mesh = plsc.VectorSubcoreMesh(core_axis_name="c", subcore_axis_name="s")


def _lane_bcast(v, k):
    idx = jnp.full((L, 1), k, jnp.int32)
    dnums = lax.GatherDimensionNumbers(
        offset_dims=(), collapsed_slice_dims=(0,), start_index_map=(0,))
    return lax.gather(v, idx, dnums, (1,),
                      mode=lax.GatherScatterMode.PROMISE_IN_BOUNDS)


@functools.partial(
    pl.kernel,
    out_type=[jax.ShapeDtypeStruct((N,), jnp.float32),
              jax.ShapeDtypeStruct((N,), jnp.float32)],
    mesh=_mesh,
    scratch_types=[
        pltpu.VMEM((CH,), jnp.int32),
        pltpu.VMEM((CH,), jnp.float32),
        pltpu.VMEM((640,), jnp.float32),
        pltpu.VMEM_SHARED((N,), jnp.float32),
    ],
)
def _deg_kernel(dst_hbm, w_hbm, out0_hbm, out1_hbm, idx_v, w_v, stage_v, deg_sh):
    cid = lax.axis_index("c")
    sid = lax.axis_index("s")
    zeros = jnp.zeros((L,), jnp.float32)
    for i in range(640 // L):
        stage_v[pl.ds(i * L, L)] = zeros
    pltpu.sync_copy(stage_v.at[pl.ds(0, 624)], deg_sh.at[pl.ds(sid * 624, 624)])

    @pl.when(sid == 15)
    def _():
        pltpu.sync_copy(stage_v.at[pl.ds(624, 16)], deg_sh.at[pl.ds(9984, 16)])

    plsc.subcore_barrier()

    ebase = (cid * NS + sid) * (DEG_CHUNKS * CH)

    def body(j, carry):
        off = ebase + j * CH
        pltpu.sync_copy(dst_hbm.at[pl.ds(off, CH)], idx_v)
        pltpu.sync_copy(w_hbm.at[pl.ds(off, CH)], w_v)
        pltpu.sync_copy(w_v, deg_sh.at[idx_v], add=True)
        return carry

    lax.fori_loop(0, DEG_CHUNKS, body, 0)
    plsc.subcore_barrier()
    for c, out_hbm in ((0, out0_hbm), (1, out1_hbm)):
        @pl.when(cid == c)
        def _(out_hbm=out_hbm):
            pltpu.sync_copy(deg_sh.at[pl.ds(sid * 624, 624)],
                            stage_v.at[pl.ds(0, 624)])
            pltpu.sync_copy(stage_v.at[pl.ds(0, 624)],
                            out_hbm.at[pl.ds(sid * 624, 624)])

            @pl.when(sid == 15)
            def _():
                pltpu.sync_copy(deg_sh.at[pl.ds(9984, 16)],
                                stage_v.at[pl.ds(624, 16)])
                pltpu.sync_copy(stage_v.at[pl.ds(624, 16)],
                                out_hbm.at[pl.ds(9984, 16)])


def _make_mp_kernel(half, edge_split):
    groups = half // L
    n_chunks = MP_CHUNKS // (NC if edge_split else 1)

    @functools.partial(
        pl.kernel,
        out_type=jax.ShapeDtypeStruct((NC, N, half), jnp.float32),
        mesh=_mesh,
        scratch_types=[
            pltpu.VMEM((CH,), jnp.int32),
            pltpu.VMEM((CH,), jnp.int32),
            pltpu.VMEM((CH,), jnp.float32),
            pltpu.VMEM((CH, half), jnp.float32),
            pltpu.VMEM_SHARED((N, half), jnp.float32),
            pltpu.SemaphoreType.DMA,
        ],
    )
    def _mp(h0_hbm, h1_hbm, src_hbm, dst_hbm, w_hbm, out_hbm,
            src_v, dst_v, w_v, rows, acc, sem):
        cid = lax.axis_index("c")
        sid = lax.axis_index("s")
        zeros = jnp.zeros((L,), jnp.float32)

        def zbody(i, carry):
            for g in range(groups):
                rows[i, pl.ds(g * L, L)] = zeros
            return carry

        lax.fori_loop(0, CH, zbody, 0)

        def zcopy(i, carry):
            pltpu.sync_copy(rows.at[pl.ds(0, 125)],
                            acc.at[pl.ds(sid * RPT + i * 125, 125)])
            return carry

        lax.fori_loop(0, RPT // 125, zcopy, 0)
        plsc.subcore_barrier()

        if edge_split:
            ebase = (cid * NS + sid) * (n_chunks * CH)
        else:
            ebase = sid * (n_chunks * CH)

        def edge_loop(h_hbm):
            def body(j, carry):
                off = ebase + j * CH
                pltpu.sync_copy(src_hbm.at[pl.ds(off, CH)], src_v)
                pltpu.sync_copy(dst_hbm.at[pl.ds(off, CH)], dst_v)
                pltpu.sync_copy(w_hbm.at[pl.ds(off, CH)], w_v)
                pltpu.async_copy(h_hbm.at[src_v], rows, sem).wait()
                for gg in range(CH // L):
                    w_grp = w_v[pl.ds(gg * L, L)]

                    def ebody(k, c2):
                        wb = _lane_bcast(w_grp, k)
                        e = gg * L + k
                        for g in range(groups):
                            s = pl.ds(g * L, L)
                            rows[e, s] = rows[e, s] * wb
                        return c2

                    lax.fori_loop(0, L, ebody, 0)
                pltpu.sync_copy(rows, acc.at[dst_v], add=True)
                return carry

            lax.fori_loop(0, n_chunks, body, 0)

        if edge_split:
            edge_loop(h0_hbm)
        else:
            @pl.when(cid == 0)
            def _():
                edge_loop(h0_hbm)

            @pl.when(cid == 1)
            def _():
                edge_loop(h1_hbm)

        plsc.subcore_barrier()

        def wcopy(i, carry):
            r0 = sid * 624 + i * 104
            pltpu.sync_copy(acc.at[pl.ds(r0, 104)], rows.at[pl.ds(0, 104)])
            pltpu.sync_copy(rows.at[pl.ds(0, 104)],
                            out_hbm.at[cid, pl.ds(r0, 104)])
            return carry

        lax.fori_loop(0, 6, wcopy, 0)

        @pl.when(sid == 15)
        def _():
            pltpu.sync_copy(acc.at[pl.ds(9984, 16)], rows.at[pl.ds(0, 16)])
            pltpu.sync_copy(rows.at[pl.ds(0, 16)],
                            out_hbm.at[cid, pl.ds(9984, 16)])

    return _mp


_mp_l1 = _make_mp_kernel(128, edge_split=False)
_mp_l2 = _make_mp_kernel(128, edge_split=True)


_R = 512
_NB = (N + _R - 1) // _R


def _mm1_body(x_ref, w1_ref, degp_ref, h_ref, dis_ref):
    deg = degp_ref[:, 0:1] + degp_ref[:, 1:2] + 1.0
    dis = lax.rsqrt(deg)
    acc = jnp.dot(x_ref[...], w1_ref[...], preferred_element_type=jnp.float32)
    h_ref[0] = acc * dis
    dis_ref[...] = dis


def _mm1(x, W1, degp):
    return pl.pallas_call(
        _mm1_body,
        grid=(_NB, NC),
        in_specs=[
            pl.BlockSpec((_R, D0), lambda r, h: (r, 0)),
            pl.BlockSpec((D0, D1 // NC), lambda r, h: (0, h)),
            pl.BlockSpec((_R, NC), lambda r, h: (r, 0)),
        ],
        out_specs=[
            pl.BlockSpec((1, _R, D1 // NC), lambda r, h: (h, r, 0)),
            pl.BlockSpec((_R, 1), lambda r, h: (r, 0)),
        ],
        out_shape=[
            jax.ShapeDtypeStruct((NC, N, D1 // NC), jnp.float32),
            jax.ShapeDtypeStruct((N, 1), jnp.float32),
        ],
        compiler_params=pltpu.CompilerParams(
            dimension_semantics=("arbitrary", "arbitrary")),
    )(x, W1, degp)


def _mm2_body(agg_a, agg_b, h_a, h_b, dis_ref, b1_ref, w2_ref, out_ref):
    dis = dis_ref[...]
    za = jnp.maximum((agg_a[0] + h_a[0]) * dis + b1_ref[:, :128], 0.0)
    zb = jnp.maximum((agg_b[0] + h_b[0]) * dis + b1_ref[:, 128:], 0.0)
    h2 = (jnp.dot(za, w2_ref[:128, :], preferred_element_type=jnp.float32)
          + jnp.dot(zb, w2_ref[128:, :], preferred_element_type=jnp.float32))
    out_ref[...] = h2 * dis


def _mm2(agg1, h1p, dis, b1, W2):
    half = D1 // NC
    return pl.pallas_call(
        _mm2_body,
        grid=(_NB,),
        in_specs=[
            pl.BlockSpec((1, _R, half), lambda r: (0, r, 0)),
            pl.BlockSpec((1, _R, half), lambda r: (1, r, 0)),
            pl.BlockSpec((1, _R, half), lambda r: (0, r, 0)),
            pl.BlockSpec((1, _R, half), lambda r: (1, r, 0)),
            pl.BlockSpec((_R, 1), lambda r: (r, 0)),
            pl.BlockSpec((1, D1), lambda r: (0, 0)),
            pl.BlockSpec((D1, D2), lambda r: (0, 0)),
        ],
        out_specs=pl.BlockSpec((_R, D2), lambda r: (r, 0)),
        out_shape=jax.ShapeDtypeStruct((N, D2), jnp.float32),
        compiler_params=pltpu.CompilerParams(
            dimension_semantics=("arbitrary",)),
    )(agg1, agg1, h1p, h1p, dis, b1, W2)


def _mm3_body(agg_a, agg_b, h_ref, dis_ref, b2_ref, out_ref):
    dis = dis_ref[...]
    out_ref[...] = jnp.maximum(
        (agg_a[0] + agg_b[0] + h_ref[...]) * dis + b2_ref[...], 0.0)


def _mm3(agg2, h2p, dis, b2):
    return pl.pallas_call(
        _mm3_body,
        grid=(_NB,),
        in_specs=[
            pl.BlockSpec((1, _R, D2), lambda r: (0, r, 0)),
            pl.BlockSpec((1, _R, D2), lambda r: (1, r, 0)),
            pl.BlockSpec((_R, D2), lambda r: (r, 0)),
            pl.BlockSpec((_R, 1), lambda r: (r, 0)),
            pl.BlockSpec((1, D2), lambda r: (0, 0)),
        ],
        out_specs=pl.BlockSpec((_R, D2), lambda r: (r, 0)),
        out_shape=jax.ShapeDtypeStruct((N, D2), jnp.float32),
        compiler_params=pltpu.CompilerParams(
            dimension_semantics=("arbitrary",)),
    )(agg2, agg2, h2p, dis, b2)


def kernel(x, edge_index, edge_weight, W1, b1, W2, b2):
    src = edge_index[0].astype(jnp.int32)
    dst = edge_index[1].astype(jnp.int32)
    w = edge_weight.astype(jnp.float32)
    npad = E_PAD - E
    src_p = jnp.concatenate([src, jnp.zeros((npad,), jnp.int32)])
    dst_p = jnp.concatenate([dst, jnp.zeros((npad,), jnp.int32)])
    w_p = jnp.concatenate([w, jnp.zeros((npad,), jnp.float32)])

    deg0, deg1 = _deg_kernel(dst_p, w_p)
    degp = jnp.stack([deg0, deg1], axis=1)
    h1p, dis = _mm1(x, W1, degp)
    agg1 = _mp_l1(h1p[0], h1p[1], src_p, dst_p, w_p)
    h2p = _mm2(agg1, h1p, dis, b1.reshape(1, D1), W2)
    agg2 = _mp_l2(h2p, h2p, src_p, dst_p, w_p)
    return _mm3(agg2, h2p, dis, b2.reshape(1, D2))


# speedup vs baseline: 7.4853x; 1.2434x over previous
import functools

import jax
import jax.numpy as jnp
from jax import lax
from jax.experimental import pallas as pl
from jax.experimental.pallas import tpu as pltpu
from jax.experimental.pallas import tpu_sc as plsc

N = 10000
E = 320000
D0, D1, D2 = 128, 256, 128

NC, NS, L = 2, 16, 16
CH = 128
E_PAD = 327680
DEG_CHUNKS = E_PAD // (NC * NS) // CH
MP_CHUNKS = E_PAD // NS // CH
RPT = N // NS
SUPER = 4

_mesh = plsc.VectorSubcoreMesh(core_axis_name="c", subcore_axis_name="s")


def _lane_bcast(v, k):
    idx = jnp.full((L, 1), k, jnp.int32)
    dnums = lax.GatherDimensionNumbers(
        offset_dims=(), collapsed_slice_dims=(0,), start_index_map=(0,))
    return lax.gather(v, idx, dnums, (1,),
                      mode=lax.GatherScatterMode.PROMISE_IN_BOUNDS)


@functools.partial(
    pl.kernel,
    out_type=[jax.ShapeDtypeStruct((N,), jnp.float32),
              jax.ShapeDtypeStruct((N,), jnp.float32)],
    mesh=_mesh,
    scratch_types=[
        pltpu.VMEM((SUPER, CH), jnp.int32),
        pltpu.VMEM((SUPER, CH), jnp.float32),
        pltpu.VMEM((640,), jnp.float32),
        pltpu.SemaphoreType.DMA,
        pltpu.VMEM_SHARED((N,), jnp.float32),
    ],
)
def _deg_kernel(dst_hbm, w_hbm, out0_hbm, out1_hbm, dst_big, w_big, stage_v,
                sem, deg_sh):
    cid = lax.axis_index("c")
    sid = lax.axis_index("s")
    zeros = jnp.zeros((L,), jnp.float32)
    for i in range(640 // L):
        stage_v[pl.ds(i * L, L)] = zeros
    pltpu.sync_copy(stage_v.at[pl.ds(0, 624)], deg_sh.at[pl.ds(sid * 624, 624)])

    @pl.when(sid == 15)
    def _():
        pltpu.sync_copy(stage_v.at[pl.ds(624, 16)], deg_sh.at[pl.ds(9984, 16)])

    plsc.subcore_barrier()

    cbase = (cid * NS + sid) * DEG_CHUNKS

    def body(j, carry):
        row0 = cbase + j * SUPER
        pltpu.sync_copy(dst_hbm.at[pl.ds(row0, SUPER)], dst_big)
        pltpu.sync_copy(w_hbm.at[pl.ds(row0, SUPER)], w_big)
        ds = [pltpu.async_copy(w_big.at[b], deg_sh.at[dst_big.at[b]],
                               sem, add=True)
              for b in range(SUPER)]
        for d in ds:
            d.wait()
        return carry

    lax.fori_loop(0, DEG_CHUNKS // SUPER, body, 0)
    plsc.subcore_barrier()
    for c, out_hbm in ((0, out0_hbm), (1, out1_hbm)):
        @pl.when(cid == c)
        def _(out_hbm=out_hbm):
            pltpu.sync_copy(deg_sh.at[pl.ds(sid * 624, 624)],
                            stage_v.at[pl.ds(0, 624)])
            pltpu.sync_copy(stage_v.at[pl.ds(0, 624)],
                            out_hbm.at[pl.ds(sid * 624, 624)])

            @pl.when(sid == 15)
            def _():
                pltpu.sync_copy(deg_sh.at[pl.ds(9984, 16)],
                                stage_v.at[pl.ds(624, 16)])
                pltpu.sync_copy(stage_v.at[pl.ds(624, 16)],
                                out_hbm.at[pl.ds(9984, 16)])


def _make_mp_kernel(half, edge_split):
    groups = half // L
    n_chunks = MP_CHUNKS // (NC if edge_split else 1)
    n_super = n_chunks // SUPER

    @functools.partial(
        pl.kernel,
        out_type=jax.ShapeDtypeStruct((NC, N, half), jnp.float32),
        mesh=_mesh,
        scratch_types=[
            pltpu.VMEM((SUPER, CH), jnp.int32),
            pltpu.VMEM((SUPER, CH), jnp.int32),
            pltpu.VMEM((SUPER, CH), jnp.float32),
            [pltpu.VMEM((CH, half), jnp.float32)] * 3,
            [pltpu.SemaphoreType.DMA] * 3,
            [pltpu.SemaphoreType.DMA] * 3,
            pltpu.VMEM_SHARED((N, half), jnp.float32),
        ],
    )
    def _mp(h0_hbm, h1_hbm, src_hbm, dst_hbm, w_hbm, out_hbm,
            src_big, dst_big, w_big, rows3, gsem, ssem, acc):
        cid = lax.axis_index("c")
        sid = lax.axis_index("s")
        rows = rows3[0]
        zeros = jnp.zeros((L,), jnp.float32)

        def zbody(i, carry):
            for g in range(groups):
                rows[i, pl.ds(g * L, L)] = zeros
            return carry

        lax.fori_loop(0, CH, zbody, 0)

        def zcopy(i, carry):
            pltpu.sync_copy(rows.at[pl.ds(0, 125)],
                            acc.at[pl.ds(sid * RPT + i * 125, 125)])
            return carry

        lax.fori_loop(0, RPT // 125, zcopy, 0)
        plsc.subcore_barrier()

        if edge_split:
            cbase = (cid * NS + sid) * n_chunks
        else:
            cbase = sid * n_chunks

        def _scale(rb, b):
            @plsc.parallel_loop(0, CH, 1, unroll=8)
            def _(e):
                g0 = (e >> 4) << 4
                w_grp = w_big[b, pl.ds(g0, L)]
                wb = _lane_bcast(w_grp, e & (L - 1))
                for g in range(groups):
                    s = pl.ds(g * L, L)
                    rb[e, s] = rb[e, s] * wb

        def edge_loop(h_hbm):
            def super_body(scj, carry):
                row0 = cbase + scj * SUPER
                pltpu.sync_copy(src_hbm.at[pl.ds(row0, SUPER)], src_big)
                pltpu.sync_copy(dst_hbm.at[pl.ds(row0, SUPER)], dst_big)
                pltpu.sync_copy(w_hbm.at[pl.ds(row0, SUPER)], w_big)
                gd = [None] * SUPER
                sd = [None] * SUPER
                for b in range(SUPER):
                    if b >= 3:
                        sd[b - 3].wait()
                    gd[b] = pltpu.async_copy(
                        h_hbm.at[src_big.at[b]], rows3[b % 3], gsem[b % 3])
                    if b >= 1:
                        gd[b - 1].wait()
                        _scale(rows3[(b - 1) % 3], b - 1)
                        sd[b - 1] = pltpu.async_copy(
                            rows3[(b - 1) % 3], acc.at[dst_big.at[b - 1]],
                            ssem[(b - 1) % 3], add=True)
                last = SUPER - 1
                gd[last].wait()
                _scale(rows3[last % 3], last)
                sd[last] = pltpu.async_copy(
                    rows3[last % 3], acc.at[dst_big.at[last]],
                    ssem[last % 3], add=True)
                for b in (SUPER - 3, SUPER - 2, SUPER - 1):
                    sd[b].wait()
                return carry

            lax.fori_loop(0, n_super, super_body, 0)

        @pl.when(cid == 0)
        def _():
            edge_loop(h0_hbm)

        @pl.when(cid == 1)
        def _():
            edge_loop(h1_hbm)

        plsc.subcore_barrier()

        def wcopy(i, carry):
            r0 = sid * 624 + i * 104
            pltpu.sync_copy(acc.at[pl.ds(r0, 104)], rows.at[pl.ds(0, 104)])
            pltpu.sync_copy(rows.at[pl.ds(0, 104)],
                            out_hbm.at[cid, pl.ds(r0, 104)])
            return carry

        lax.fori_loop(0, 6, wcopy, 0)

        @pl.when(sid == 15)
        def _():
            pltpu.sync_copy(acc.at[pl.ds(9984, 16)], rows.at[pl.ds(0, 16)])
            pltpu.sync_copy(rows.at[pl.ds(0, 16)],
                            out_hbm.at[cid, pl.ds(9984, 16)])

    return _mp


_mp_l1 = _make_mp_kernel(128, edge_split=False)
_mp_l2 = _make_mp_kernel(128, edge_split=True)


_R = 512
_NB = (N + _R - 1) // _R


def _mm1_body(x_ref, w1_ref, degp_ref, h_ref, dis_ref):
    deg = degp_ref[:, 0:1] + degp_ref[:, 1:2] + 1.0
    dis = lax.rsqrt(deg)
    acc = jnp.dot(x_ref[...], w1_ref[...], preferred_element_type=jnp.float32)
    h_ref[...] = acc * dis
    dis_ref[...] = dis


def _mm1(x, Wh, degp):
    return pl.pallas_call(
        _mm1_body,
        grid=(_NB,),
        in_specs=[
            pl.BlockSpec((_R, D0), lambda r: (r, 0)),
            pl.BlockSpec((D0, D1 // NC), lambda r: (0, 0)),
            pl.BlockSpec((_R, NC), lambda r: (r, 0)),
        ],
        out_specs=[
            pl.BlockSpec((_R, D1 // NC), lambda r: (r, 0)),
            pl.BlockSpec((_R, 1), lambda r: (r, 0)),
        ],
        out_shape=[
            jax.ShapeDtypeStruct((N, D1 // NC), jnp.float32),
            jax.ShapeDtypeStruct((N, 1), jnp.float32),
        ],
        compiler_params=pltpu.CompilerParams(
            dimension_semantics=("arbitrary",)),
    )(x, Wh, degp)


def _mm2_body(agg_a, agg_b, h_a, h_b, dis_ref, b1_ref, w2_ref,
              out0_ref, out1_ref):
    dis = dis_ref[...]
    za = jnp.maximum((agg_a[0] + h_a[...]) * dis + b1_ref[:, :128], 0.0)
    zb = jnp.maximum((agg_b[0] + h_b[...]) * dis + b1_ref[:, 128:], 0.0)
    h2 = (jnp.dot(za, w2_ref[:128, :], preferred_element_type=jnp.float32)
          + jnp.dot(zb, w2_ref[128:, :], preferred_element_type=jnp.float32))
    h2p = h2 * dis
    out0_ref[...] = h2p
    out1_ref[...] = h2p


def _mm2(agg1, h1a, h1b, dis, b1, W2):
    half = D1 // NC
    return pl.pallas_call(
        _mm2_body,
        grid=(_NB,),
        in_specs=[
            pl.BlockSpec((1, _R, half), lambda r: (0, r, 0)),
            pl.BlockSpec((1, _R, half), lambda r: (1, r, 0)),
            pl.BlockSpec((_R, half), lambda r: (r, 0)),
            pl.BlockSpec((_R, half), lambda r: (r, 0)),
            pl.BlockSpec((_R, 1), lambda r: (r, 0)),
            pl.BlockSpec((1, D1), lambda r: (0, 0)),
            pl.BlockSpec((D1, D2), lambda r: (0, 0)),
        ],
        out_specs=[
            pl.BlockSpec((_R, D2), lambda r: (r, 0)),
            pl.BlockSpec((_R, D2), lambda r: (r, 0)),
        ],
        out_shape=[
            jax.ShapeDtypeStruct((N, D2), jnp.float32),
            jax.ShapeDtypeStruct((N, D2), jnp.float32),
        ],
        compiler_params=pltpu.CompilerParams(
            dimension_semantics=("arbitrary",)),
    )(agg1, agg1, h1a, h1b, dis, b1, W2)


def _mm3_body(agg_a, agg_b, h_ref, dis_ref, b2_ref, out_ref):
    dis = dis_ref[...]
    out_ref[...] = jnp.maximum(
        (agg_a[0] + agg_b[0] + h_ref[...]) * dis + b2_ref[...], 0.0)


def _mm3(agg2, h2p, dis, b2):
    return pl.pallas_call(
        _mm3_body,
        grid=(_NB,),
        in_specs=[
            pl.BlockSpec((1, _R, D2), lambda r: (0, r, 0)),
            pl.BlockSpec((1, _R, D2), lambda r: (1, r, 0)),
            pl.BlockSpec((_R, D2), lambda r: (r, 0)),
            pl.BlockSpec((_R, 1), lambda r: (r, 0)),
            pl.BlockSpec((1, D2), lambda r: (0, 0)),
        ],
        out_specs=pl.BlockSpec((_R, D2), lambda r: (r, 0)),
        out_shape=jax.ShapeDtypeStruct((N, D2), jnp.float32),
        compiler_params=pltpu.CompilerParams(
            dimension_semantics=("arbitrary",)),
    )(agg2, agg2, h2p, dis, b2)


def kernel(x, edge_index, edge_weight, W1, b1, W2, b2):
    src = edge_index[0].astype(jnp.int32)
    dst = edge_index[1].astype(jnp.int32)
    w = edge_weight.astype(jnp.float32)
    npad = E_PAD - E
    nrows = E_PAD // CH
    src_p = jnp.concatenate([src, jnp.zeros((npad,), jnp.int32)]).reshape(
        nrows, CH)
    dst_p = jnp.concatenate([dst, jnp.zeros((npad,), jnp.int32)]).reshape(
        nrows, CH)
    w_p = jnp.concatenate([w, jnp.zeros((npad,), jnp.float32)]).reshape(
        nrows, CH)

    deg0, deg1 = _deg_kernel(dst_p, w_p)
    degp = jnp.stack([deg0, deg1], axis=1)
    h1a, dis = _mm1(x, W1[:, :D1 // NC], degp)
    h1b, _ = _mm1(x, W1[:, D1 // NC:], degp)
    agg1 = _mp_l1(h1a, h1b, src_p, dst_p, w_p)
    h2p0, h2p1 = _mm2(agg1, h1a, h1b, dis, b1.reshape(1, D1), W2)
    agg2 = _mp_l2(h2p0, h2p1, src_p, dst_p, w_p)
    return _mm3(agg2, h2p0, dis, b2.reshape(1, D2))
